# baseline (speedup 1.0000x reference)
import functools

import jax
import jax.numpy as jnp
from jax import lax
from jax.experimental import pallas as pl
from jax.experimental.pallas import tpu as pltpu
from jax.experimental.pallas import tpu_sc as plsc

CODEBOOK = 1024
EMB = 256
B = 8
TOK = 1024
BETA = 0.25
N_TOKENS = B * TOK
N_ELEMS = N_TOKENS * EMB

NC = 2
NS = 16
NW = NC * NS
CH_PER_W = EMB // NW
TOK_PER_W = N_TOKENS // NW
L = 16


def _dist_body(z_ref, w_ref, idx_ref, z2s_ref, sqs_ref, zrs_ref):
    s = pl.program_id(0)
    zb = z_ref[0]
    w = w_ref[...]

    w2x = w + w
    m2 = jax.lax.dot_general(w2x, zb, (((1,), (0,)), ((), ())),
                             preferred_element_type=jnp.float32)
    z2 = jnp.sum(zb * zb, axis=0, keepdims=True)
    w2 = jnp.sum(w * w, axis=1, keepdims=True)
    d = (z2 + w2) - m2

    minv = jnp.min(d, axis=0, keepdims=True)
    ci = jax.lax.broadcasted_iota(jnp.int32, (CODEBOOK, TOK), 0)
    big = jnp.int32(1 << 30)
    idx = jnp.min(jnp.where(d == minv, ci, big), axis=0, keepdims=True)
    idx_ref[0] = idx

    z2s = jnp.sum(z2)
    sq = jnp.sum(minv)
    zrs = jnp.sum(zb, axis=1, keepdims=True)

    @pl.when(s == 0)
    def _init():
        z2s_ref[0, 0] = z2s
        sqs_ref[0, 0] = sq
        zrs_ref[...] = zrs

    @pl.when(s > 0)
    def _acc():
        z2s_ref[0, 0] += z2s
        sqs_ref[0, 0] += sq
        zrs_ref[...] += zrs


def _sc_gather_body(wt_hbm, idx_hbm, out_hbm, hist_hbm,
                    wt_v, idx_v, outbuf, hist2d, histsum):
    c = lax.axis_index("c")
    s = lax.axis_index("s")
    wid = s * NC + c
    ch0 = wid * CH_PER_W

    pltpu.sync_copy(wt_hbm.at[pl.ds(ch0 * CODEBOOK, CH_PER_W * CODEBOOK)], wt_v)
    pltpu.sync_copy(idx_hbm, idx_v)

    zeros16 = jnp.zeros((L,), jnp.float32)
    ones16 = jnp.ones((L,), jnp.float32)
    lane = lax.iota(jnp.int32, L)

    def _zero(j, carry):
        hist2d[pl.ds(j * L, L)] = zeros16
        return carry
    lax.fori_loop(0, L * CODEBOOK // L, _zero, 0)

    def _hist(j, carry):
        iv = idx_v[pl.ds(wid * TOK_PER_W + j * L, L)]
        plsc.addupdate_scatter(hist2d, [lane * jnp.int32(CODEBOOK) + iv], ones16)
        return carry
    lax.fori_loop(0, TOK_PER_W // L, _hist, 0)

    def _hsum(j, carry):
        acc = hist2d[pl.ds(j * L, L)]
        for r in range(1, L):
            acc = acc + hist2d[pl.ds(r * CODEBOOK + j * L, L)]
        histsum[pl.ds(j * L, L)] = acc
        return carry
    lax.fori_loop(0, CODEBOOK // L, _hsum, 0)
    pltpu.sync_copy(histsum, hist_hbm.at[pl.ds(wid * CODEBOOK, CODEBOOK)])

    for b in range(B):
        def _gather(j, carry):
            iv = idx_v[pl.ds(b * TOK + j * L, L)]
            for k in range(CH_PER_W):
                row = plsc.load_gather(wt_v, [iv + jnp.int32(k * CODEBOOK)])
                outbuf[pl.ds(k * TOK + j * L, L)] = row
            return carry
        lax.fori_loop(0, TOK // L, _gather, 0)
        pltpu.sync_copy(outbuf,
                        out_hbm.at[pl.ds(b * EMB * TOK + ch0 * TOK,
                                         CH_PER_W * TOK)])


def _final_body(hist_ref, w_ref, zrs_ref, z2s_ref, sqs_ref,
                loss_ref, perp_ref, md_ref):
    w = w_ref[...]
    counts = jnp.sum(hist_ref[...], axis=0, keepdims=True)
    e = counts / jnp.float32(N_TOKENS)
    ent = jnp.sum(e * jnp.log(e + jnp.float32(1e-10)))
    perp_ref[0, 0] = jnp.exp(-ent)

    msq = sqs_ref[0, 0] / jnp.float32(N_ELEMS)
    loss_ref[0, 0] = jnp.float32(BETA) * msq + msq

    w2s = jnp.sum(w * w)
    wcs = jnp.sum(w + w, axis=0, keepdims=True)
    m2s = jax.lax.dot_general(wcs, zrs_ref[...], (((1,), (0,)), ((), ())),
                              preferred_element_type=jnp.float32)
    dsum = (jnp.float32(CODEBOOK) * (z2s_ref[0, 0]
                                     + jnp.float32(N_TOKENS) * w2s)
            - m2s[0, 0])
    md_ref[0, 0] = dsum / jnp.float32(N_TOKENS * CODEBOOK)


@functools.partial(jax.jit, static_argnames=("interpret",))
def kernel(z, W, interpret=False):
    z3 = z.reshape(B, EMB, TOK)

    idx, z2s, sqs, zrs = pl.pallas_call(
        _dist_body,
        grid=(B,),
        in_specs=[
            pl.BlockSpec((1, EMB, TOK), lambda s: (s, 0, 0)),
            pl.BlockSpec((CODEBOOK, EMB), lambda s: (0, 0)),
        ],
        out_specs=(
            pl.BlockSpec((1, 1, TOK), lambda s: (s, 0, 0)),
            pl.BlockSpec(memory_space=pltpu.SMEM),
            pl.BlockSpec(memory_space=pltpu.SMEM),
            pl.BlockSpec((EMB, 1), lambda s: (0, 0)),
        ),
        out_shape=(
            jax.ShapeDtypeStruct((B, 1, TOK), jnp.int32),
            jax.ShapeDtypeStruct((1, 1), jnp.float32),
            jax.ShapeDtypeStruct((1, 1), jnp.float32),
            jax.ShapeDtypeStruct((EMB, 1), jnp.float32),
        ),
        interpret=interpret,
    )(z3, W)

    idxf = idx.reshape(N_TOKENS)
    mesh = plsc.VectorSubcoreMesh(core_axis_name="c", subcore_axis_name="s",
                                  num_cores=NC, num_subcores=NS)
    zq, hist = pl.kernel(
        _sc_gather_body,
        out_type=(
            jax.ShapeDtypeStruct((B * EMB * TOK,), jnp.float32),
            jax.ShapeDtypeStruct((NW * CODEBOOK,), jnp.float32),
        ),
        mesh=mesh,
        scratch_types=(
            pltpu.VMEM((EMB * CODEBOOK // NW,), jnp.float32),
            pltpu.VMEM((N_TOKENS,), jnp.int32),
            pltpu.VMEM((CH_PER_W * TOK,), jnp.float32),
            pltpu.VMEM((L * CODEBOOK,), jnp.float32),
            pltpu.VMEM((CODEBOOK,), jnp.float32),
        ),
        compiler_params=pltpu.CompilerParams(needs_layout_passes=False),
        interpret=interpret,
    )(W.T.reshape(-1), idxf)
    hist = hist.reshape(NW, CODEBOOK)

    loss, perp, md = pl.pallas_call(
        _final_body,
        grid=(1,),
        in_specs=[
            pl.BlockSpec((NW, CODEBOOK), lambda i: (0, 0)),
            pl.BlockSpec((CODEBOOK, EMB), lambda i: (0, 0)),
            pl.BlockSpec((EMB, 1), lambda i: (0, 0)),
            pl.BlockSpec(memory_space=pltpu.SMEM),
            pl.BlockSpec(memory_space=pltpu.SMEM),
        ],
        out_specs=(
            pl.BlockSpec(memory_space=pltpu.SMEM),
            pl.BlockSpec(memory_space=pltpu.SMEM),
            pl.BlockSpec(memory_space=pltpu.SMEM),
        ),
        out_shape=(
            jax.ShapeDtypeStruct((1, 1), jnp.float32),
            jax.ShapeDtypeStruct((1, 1), jnp.float32),
            jax.ShapeDtypeStruct((1, 1), jnp.float32),
        ),
        interpret=interpret,
    )(hist, W, zrs, z2s, sqs)

    z_q = zq.reshape(B, EMB, 32, 32)
    min_encoding_indices = idxf.reshape(N_TOKENS, 1)
    return (z_q, loss[0, 0], perp[0, 0], md[0, 0], min_encoding_indices)


# speedup vs baseline: 2.4923x; 2.4923x over previous
import functools

import jax
import jax.numpy as jnp
from jax.experimental import pallas as pl
from jax.experimental.pallas import tpu as pltpu

CODEBOOK = 1024
EMB = 256
B = 8
TOK = 1024
BPS = 1
TOKS = TOK * BPS
STEPS = B // BPS
BETA = 0.25
N_TOKENS = B * TOK
N_ELEMS = N_TOKENS * EMB


def _vq_body(z_ref, w_ref,
             zq_ref, idx_ref, loss_ref, perp_ref, md_ref,
             z2sum_acc, sqsum_acc, counts_acc, zrs_acc):
    s = pl.program_id(0)
    zb = z_ref[0]
    w = w_ref[...]

    w2x = w + w
    m2 = jax.lax.dot_general(w2x, zb, (((1,), (0,)), ((), ())),
                             preferred_element_type=jnp.float32)
    z2 = jnp.sum(zb * zb, axis=0, keepdims=True)
    w2 = jnp.sum(w * w, axis=1, keepdims=True)
    d = (z2 + w2) - m2

    minv = jnp.min(d, axis=0, keepdims=True)
    ci = jax.lax.broadcasted_iota(jnp.int32, (CODEBOOK, TOKS), 0)
    big = jnp.int32(1 << 30)
    idx = jnp.min(jnp.where(d == minv, ci, big), axis=0, keepdims=True)
    idx_ref[0] = idx

    onehot = (ci == idx).astype(jnp.float32)
    zq_ref[0] = jax.lax.dot_general(w, onehot, (((0,), (0,)), ((), ())),
                                    preferred_element_type=jnp.float32)

    z2s = jnp.sum(z2)
    zrs = jnp.sum(zb, axis=1, keepdims=True)
    sq = jnp.sum(minv)
    cnt = jnp.sum(onehot, axis=1, keepdims=True)

    @pl.when(s == 0)
    def _init():
        z2sum_acc[0, 0] = z2s
        sqsum_acc[0, 0] = sq
        counts_acc[...] = cnt
        zrs_acc[...] = zrs

    @pl.when(s > 0)
    def _acc():
        z2sum_acc[0, 0] += z2s
        sqsum_acc[0, 0] += sq
        counts_acc[...] += cnt
        zrs_acc[...] += zrs

    @pl.when(s == STEPS - 1)
    def _fin():
        wcs = jnp.sum(w2x, axis=0, keepdims=True)
        m2s = jax.lax.dot_general(wcs, zrs_acc[...], (((1,), (0,)), ((), ())),
                                  preferred_element_type=jnp.float32)
        dsum = (jnp.float32(CODEBOOK) * (z2sum_acc[0, 0]
                                         + jnp.float32(N_TOKENS) * jnp.sum(w2))
                - m2s[0, 0])
        md_ref[0, 0] = dsum / jnp.float32(N_TOKENS * CODEBOOK)
        msq = sqsum_acc[0, 0] / jnp.float32(N_ELEMS)
        loss_ref[0, 0] = jnp.float32(BETA) * msq + msq
        e = counts_acc[...] / jnp.float32(N_TOKENS)
        ent = jnp.sum(e * jnp.log(e + jnp.float32(1e-10)))
        perp_ref[0, 0] = jnp.exp(-ent)


@functools.partial(jax.jit, static_argnames=("interpret",))
def kernel(z, W, interpret=False):
    z3 = z.reshape(STEPS, EMB, TOKS)
    grid = (STEPS,)
    out_shapes = (
        jax.ShapeDtypeStruct((STEPS, EMB, TOKS), jnp.float32),
        jax.ShapeDtypeStruct((STEPS, 1, TOKS), jnp.int32),
        jax.ShapeDtypeStruct((1, 1), jnp.float32),
        jax.ShapeDtypeStruct((1, 1), jnp.float32),
        jax.ShapeDtypeStruct((1, 1), jnp.float32),
    )
    zq, idx, loss, perp, md = pl.pallas_call(
        _vq_body,
        grid=grid,
        in_specs=[
            pl.BlockSpec((1, EMB, TOKS), lambda s: (s, 0, 0)),
            pl.BlockSpec((CODEBOOK, EMB), lambda s: (0, 0)),
        ],
        out_specs=(
            pl.BlockSpec((1, EMB, TOKS), lambda s: (s, 0, 0)),
            pl.BlockSpec((1, 1, TOKS), lambda s: (s, 0, 0)),
            pl.BlockSpec(memory_space=pltpu.SMEM),
            pl.BlockSpec(memory_space=pltpu.SMEM),
            pl.BlockSpec(memory_space=pltpu.SMEM),
        ),
        out_shape=out_shapes,
        scratch_shapes=[
            pltpu.SMEM((1, 1), jnp.float32),
            pltpu.SMEM((1, 1), jnp.float32),
            pltpu.VMEM((CODEBOOK, 1), jnp.float32),
            pltpu.VMEM((EMB, 1), jnp.float32),
        ],
        compiler_params=pltpu.CompilerParams(fuse_transposed_lhs_in_matmul=True),
        interpret=interpret,
    )(z3, W)

    z_q = zq.reshape(B, EMB, 32, 32)
    min_encoding_indices = idx.reshape(N_TOKENS, 1)
    return (z_q, loss[0, 0], perp[0, 0], md[0, 0], min_encoding_indices)


# speedup vs baseline: 2.7641x; 1.1090x over previous
import functools

import jax
import jax.numpy as jnp
from jax.experimental import pallas as pl
from jax.experimental.pallas import tpu as pltpu

CODEBOOK = 1024
EMB = 256
B = 8
TOK = 1024
BPS = 1
TOKS = TOK * BPS
STEPS = B // BPS
BETA = 0.25
N_TOKENS = B * TOK
N_ELEMS = N_TOKENS * EMB


def _vq_body(z_ref, w_ref,
             zq_ref, idx_ref, loss_ref, perp_ref, md_ref,
             z2sum_acc, sqsum_acc, counts_acc, zrs_acc):
    s = pl.program_id(0)
    zb = z_ref[0]
    w = w_ref[...]

    w2x = w + w
    m2 = jax.lax.dot_general(w2x, zb, (((1,), (0,)), ((), ())),
                             preferred_element_type=jnp.float32)
    z2 = jnp.sum(zb * zb, axis=0, keepdims=True)
    w2 = jnp.sum(w * w, axis=1, keepdims=True)
    d = (z2 + w2) - m2

    minv = jnp.min(d, axis=0, keepdims=True)
    idx = jnp.argmin(d, axis=0).astype(jnp.int32)[None, :]
    idx_ref[0] = idx

    ci = jax.lax.broadcasted_iota(jnp.int32, (CODEBOOK, TOKS), 0)
    onehot = (ci == idx).astype(jnp.float32)
    zq_ref[0] = jax.lax.dot_general(w, onehot, (((0,), (0,)), ((), ())),
                                    preferred_element_type=jnp.float32)

    z2s = jnp.sum(z2)
    zrs = jnp.sum(zb, axis=1, keepdims=True)
    sq = jnp.sum(minv)
    cnt = jnp.sum(onehot, axis=1, keepdims=True)

    @pl.when(s == 0)
    def _init():
        z2sum_acc[0, 0] = z2s
        sqsum_acc[0, 0] = sq
        counts_acc[...] = cnt
        zrs_acc[...] = zrs

    @pl.when(s > 0)
    def _acc():
        z2sum_acc[0, 0] += z2s
        sqsum_acc[0, 0] += sq
        counts_acc[...] += cnt
        zrs_acc[...] += zrs

    @pl.when(s == STEPS - 1)
    def _fin():
        wcs = jnp.sum(w2x, axis=0, keepdims=True)
        m2s = jax.lax.dot_general(wcs, zrs_acc[...], (((1,), (0,)), ((), ())),
                                  preferred_element_type=jnp.float32)
        dsum = (jnp.float32(CODEBOOK) * (z2sum_acc[0, 0]
                                         + jnp.float32(N_TOKENS) * jnp.sum(w2))
                - m2s[0, 0])
        md_ref[0, 0] = dsum / jnp.float32(N_TOKENS * CODEBOOK)
        msq = sqsum_acc[0, 0] / jnp.float32(N_ELEMS)
        loss_ref[0, 0] = jnp.float32(BETA) * msq + msq
        e = counts_acc[...] / jnp.float32(N_TOKENS)
        ent = jnp.sum(e * jnp.log(e + jnp.float32(1e-10)))
        perp_ref[0, 0] = jnp.exp(-ent)


@functools.partial(jax.jit, static_argnames=("interpret",))
def kernel(z, W, interpret=False):
    z3 = z.reshape(STEPS, EMB, TOKS)
    grid = (STEPS,)
    out_shapes = (
        jax.ShapeDtypeStruct((STEPS, EMB, TOKS), jnp.float32),
        jax.ShapeDtypeStruct((STEPS, 1, TOKS), jnp.int32),
        jax.ShapeDtypeStruct((1, 1), jnp.float32),
        jax.ShapeDtypeStruct((1, 1), jnp.float32),
        jax.ShapeDtypeStruct((1, 1), jnp.float32),
    )
    zq, idx, loss, perp, md = pl.pallas_call(
        _vq_body,
        grid=grid,
        in_specs=[
            pl.BlockSpec((1, EMB, TOKS), lambda s: (s, 0, 0)),
            pl.BlockSpec((CODEBOOK, EMB), lambda s: (0, 0)),
        ],
        out_specs=(
            pl.BlockSpec((1, EMB, TOKS), lambda s: (s, 0, 0)),
            pl.BlockSpec((1, 1, TOKS), lambda s: (s, 0, 0)),
            pl.BlockSpec(memory_space=pltpu.SMEM),
            pl.BlockSpec(memory_space=pltpu.SMEM),
            pl.BlockSpec(memory_space=pltpu.SMEM),
        ),
        out_shape=out_shapes,
        scratch_shapes=[
            pltpu.SMEM((1, 1), jnp.float32),
            pltpu.SMEM((1, 1), jnp.float32),
            pltpu.VMEM((CODEBOOK, 1), jnp.float32),
            pltpu.VMEM((EMB, 1), jnp.float32),
        ],
        interpret=interpret,
    )(z3, W)

    z_q = zq.reshape(B, EMB, 32, 32)
    min_encoding_indices = idx.reshape(N_TOKENS, 1)
    return (z_q, loss[0, 0], perp[0, 0], md[0, 0], min_encoding_indices)
